# baseline (device time: 46716 ns/iter reference)
import jax
import jax.numpy as jnp
from jax import lax
from jax.experimental import pallas as pl
from jax.experimental.pallas import tpu as pltpu

N_DEV = 4


def kernel(x, w_mat):
    m_full, _ = x.shape
    _, n = w_mat.shape
    m_chunk = m_full // N_DEV

    def body(x_ref, w_ref, out_ref, p_ref, send_ref, recv_ref,
             send_sems, recv_sems):
        my = lax.axis_index("i")
        left = lax.rem(my + N_DEV - 1, N_DEV)
        right = lax.rem(my + 1, N_DEV)

        barrier_sem = pltpu.get_barrier_semaphore()
        for nbr in (left, right):
            pl.semaphore_signal(
                barrier_sem, inc=1,
                device_id=(nbr,), device_id_type=pl.DeviceIdType.MESH,
            )
        pl.semaphore_wait(barrier_sem, 2)

        p_ref[:, :] = jnp.dot(
            x_ref[:, :], w_ref[:, :], preferred_element_type=jnp.float32
        )

        for s in range(N_DEV - 1):
            c = lax.rem(my - (s + 1) + 2 * N_DEV, N_DEV)
            chunk = p_ref[pl.ds(c * m_chunk, m_chunk), :]
            if s == 0:
                acc = chunk
            else:
                acc = chunk + recv_ref[s - 1]
            send_ref[s, :, :] = acc
            rdma = pltpu.make_async_remote_copy(
                src_ref=send_ref.at[s],
                dst_ref=recv_ref.at[s],
                send_sem=send_sems.at[s],
                recv_sem=recv_sems.at[s],
                device_id=(right,),
                device_id_type=pl.DeviceIdType.MESH,
            )
            rdma.start()
            rdma.wait()

        out_ref[:, :] = (
            p_ref[pl.ds(my * m_chunk, m_chunk), :] + recv_ref[N_DEV - 2]
        )

    return pl.pallas_call(
        body,
        out_shape=jax.ShapeDtypeStruct((m_chunk, n), jnp.float32),
        in_specs=[
            pl.BlockSpec(memory_space=pltpu.VMEM),
            pl.BlockSpec(memory_space=pltpu.VMEM),
        ],
        out_specs=pl.BlockSpec(memory_space=pltpu.VMEM),
        scratch_shapes=[
            pltpu.VMEM((m_full, n), jnp.float32),
            pltpu.VMEM((N_DEV - 1, m_chunk, n), jnp.float32),
            pltpu.VMEM((N_DEV - 1, m_chunk, n), jnp.float32),
            pltpu.SemaphoreType.DMA((N_DEV - 1,)),
            pltpu.SemaphoreType.DMA((N_DEV - 1,)),
        ],
        compiler_params=pltpu.CompilerParams(collective_id=0),
    )(x, w_mat)


# device time: 29324 ns/iter; 1.5931x vs baseline; 1.5931x over previous
import jax
import jax.numpy as jnp
from jax import lax
from jax.experimental import pallas as pl
from jax.experimental.pallas import tpu as pltpu

N_DEV = 4


def kernel(x, w_mat):
    m_full, _ = x.shape
    _, n = w_mat.shape
    m_chunk = m_full // N_DEV
    nh = n // 2

    def body(x_ref, w_ref, out_ref, p_ref,
             send_r, recv_r, send_l, recv_l,
             ssem_r, rsem_r, ssem_l, rsem_l):
        my = lax.axis_index("i")
        left = lax.rem(my + N_DEV - 1, N_DEV)
        right = lax.rem(my + 1, N_DEV)

        def rows(c):
            return pl.ds(lax.rem(c + 2 * N_DEV, N_DEV) * m_chunk, m_chunk)

        def rdma(src, dst, ssem, rsem, dev):
            return pltpu.make_async_remote_copy(
                src_ref=src, dst_ref=dst, send_sem=ssem, recv_sem=rsem,
                device_id=(dev,), device_id_type=pl.DeviceIdType.MESH,
            )

        barrier_sem = pltpu.get_barrier_semaphore()
        for nbr in (left, right):
            pl.semaphore_signal(
                barrier_sem, inc=1,
                device_id=(nbr,), device_id_type=pl.DeviceIdType.MESH,
            )
        pl.semaphore_wait(barrier_sem, 2)

        descs = []

        send_r[0, :, :] = jnp.dot(
            x_ref[rows(my - 1), :], w_ref[:, :nh],
            preferred_element_type=jnp.float32,
        )
        r0 = rdma(send_r.at[0], recv_r.at[0], ssem_r.at[0], rsem_r.at[0], right)
        r0.start()
        send_l[0, :, :] = jnp.dot(
            x_ref[rows(my + 1), :], w_ref[:, nh:],
            preferred_element_type=jnp.float32,
        )
        l0 = rdma(send_l.at[0], recv_l.at[0], ssem_l.at[0], rsem_l.at[0], left)
        l0.start()
        descs += [r0, l0]

        p_ref[rows(my + 2), :] = jnp.dot(
            x_ref[rows(my + 2), :], w_ref[:, :],
            preferred_element_type=jnp.float32,
        )

        r0.wait_recv()
        send_r[1, :, :] = p_ref[rows(my + 2), :nh] + recv_r[0]
        r1 = rdma(send_r.at[1], recv_r.at[1], ssem_r.at[1], rsem_r.at[1], right)
        r1.start()
        l0.wait_recv()
        send_l[1, :, :] = p_ref[rows(my + 2), nh:] + recv_l[0]
        l1 = rdma(send_l.at[1], recv_l.at[1], ssem_l.at[1], rsem_l.at[1], left)
        l1.start()
        descs += [r1, l1]

        p_ref[rows(my + 1), :nh] = jnp.dot(
            x_ref[rows(my + 1), :], w_ref[:, :nh],
            preferred_element_type=jnp.float32,
        )
        p_ref[rows(my - 1), nh:] = jnp.dot(
            x_ref[rows(my - 1), :], w_ref[:, nh:],
            preferred_element_type=jnp.float32,
        )
        p_ref[rows(my), :] = jnp.dot(
            x_ref[rows(my), :], w_ref[:, :],
            preferred_element_type=jnp.float32,
        )

        r1.wait_recv()
        send_r[2, :, :] = p_ref[rows(my + 1), :nh] + recv_r[1]
        r2 = rdma(send_r.at[2], recv_r.at[2], ssem_r.at[2], rsem_r.at[2], right)
        r2.start()
        l1.wait_recv()
        send_l[2, :, :] = p_ref[rows(my - 1), nh:] + recv_l[1]
        l2 = rdma(send_l.at[2], recv_l.at[2], ssem_l.at[2], rsem_l.at[2], left)
        l2.start()
        descs += [r2, l2]

        r2.wait_recv()
        out_ref[:, :nh] = p_ref[rows(my), :nh] + recv_r[2]
        l2.wait_recv()
        out_ref[:, nh:] = p_ref[rows(my), nh:] + recv_l[2]

        for d in descs:
            d.wait_send()

    return pl.pallas_call(
        body,
        out_shape=jax.ShapeDtypeStruct((m_chunk, n), jnp.float32),
        in_specs=[
            pl.BlockSpec(memory_space=pltpu.VMEM),
            pl.BlockSpec(memory_space=pltpu.VMEM),
        ],
        out_specs=pl.BlockSpec(memory_space=pltpu.VMEM),
        scratch_shapes=[
            pltpu.VMEM((m_full, n), jnp.float32),
            pltpu.VMEM((N_DEV - 1, m_chunk, nh), jnp.float32),
            pltpu.VMEM((N_DEV - 1, m_chunk, nh), jnp.float32),
            pltpu.VMEM((N_DEV - 1, m_chunk, nh), jnp.float32),
            pltpu.VMEM((N_DEV - 1, m_chunk, nh), jnp.float32),
            pltpu.SemaphoreType.DMA((N_DEV - 1,)),
            pltpu.SemaphoreType.DMA((N_DEV - 1,)),
            pltpu.SemaphoreType.DMA((N_DEV - 1,)),
            pltpu.SemaphoreType.DMA((N_DEV - 1,)),
        ],
        compiler_params=pltpu.CompilerParams(collective_id=0),
    )(x, w_mat)


# device time: 25846 ns/iter; 1.8075x vs baseline; 1.1346x over previous
import jax
import jax.numpy as jnp
from jax import lax
from jax.experimental import pallas as pl
from jax.experimental.pallas import tpu as pltpu

N_DEV = 4
SEG = 2


def kernel(x, w_mat):
    m_full, _ = x.shape
    _, n = w_mat.shape
    m_chunk = m_full // N_DEV
    nh = n // 2
    segw = nh // SEG

    def body(x_ref, w_ref, out_ref, p_ref,
             send_r, recv_r, send_l, recv_l,
             ssem_r, rsem_r, ssem_l, rsem_l):
        my = lax.axis_index("i")
        left = lax.rem(my + N_DEV - 1, N_DEV)
        right = lax.rem(my + 1, N_DEV)

        def rows(c):
            return pl.ds(lax.rem(c + 2 * N_DEV, N_DEV) * m_chunk, m_chunk)

        def colA(k):
            return slice(k * segw, (k + 1) * segw)

        def colB(k):
            return slice(nh + k * segw, nh + (k + 1) * segw)

        def rdma(buf_s, buf_r, ssem, rsem, s, k, dev):
            return pltpu.make_async_remote_copy(
                src_ref=buf_s.at[s, k], dst_ref=buf_r.at[s, k],
                send_sem=ssem.at[s, k], recv_sem=rsem.at[s, k],
                device_id=(dev,), device_id_type=pl.DeviceIdType.MESH,
            )

        barrier_sem = pltpu.get_barrier_semaphore()
        for nbr in (left, right):
            pl.semaphore_signal(
                barrier_sem, inc=1,
                device_id=(nbr,), device_id_type=pl.DeviceIdType.MESH,
            )
        pl.semaphore_wait(barrier_sem, 2)

        descs_r = {}
        descs_l = {}

        for k in range(SEG):
            send_r[0, k, :, :] = jnp.dot(
                x_ref[rows(my - 1), :], w_ref[:, colA(k)],
                preferred_element_type=jnp.float32,
            )
            d = rdma(send_r, recv_r, ssem_r, rsem_r, 0, k, right)
            d.start()
            descs_r[0, k] = d
            send_l[0, k, :, :] = jnp.dot(
                x_ref[rows(my + 1), :], w_ref[:, colB(k)],
                preferred_element_type=jnp.float32,
            )
            d = rdma(send_l, recv_l, ssem_l, rsem_l, 0, k, left)
            d.start()
            descs_l[0, k] = d

        p_ref[rows(my + 2), :] = jnp.dot(
            x_ref[rows(my + 2), :], w_ref[:, :],
            preferred_element_type=jnp.float32,
        )

        for k in range(SEG):
            descs_r[0, k].wait_recv()
            send_r[1, k, :, :] = p_ref[rows(my + 2), colA(k)] + recv_r[0, k]
            d = rdma(send_r, recv_r, ssem_r, rsem_r, 1, k, right)
            d.start()
            descs_r[1, k] = d
            descs_l[0, k].wait_recv()
            send_l[1, k, :, :] = p_ref[rows(my + 2), colB(k)] + recv_l[0, k]
            d = rdma(send_l, recv_l, ssem_l, rsem_l, 1, k, left)
            d.start()
            descs_l[1, k] = d

        p_ref[rows(my + 1), :nh] = jnp.dot(
            x_ref[rows(my + 1), :], w_ref[:, :nh],
            preferred_element_type=jnp.float32,
        )
        p_ref[rows(my - 1), nh:] = jnp.dot(
            x_ref[rows(my - 1), :], w_ref[:, nh:],
            preferred_element_type=jnp.float32,
        )
        p_ref[rows(my), :] = jnp.dot(
            x_ref[rows(my), :], w_ref[:, :],
            preferred_element_type=jnp.float32,
        )

        for k in range(SEG):
            descs_r[1, k].wait_recv()
            send_r[2, k, :, :] = p_ref[rows(my + 1), colA(k)] + recv_r[1, k]
            d = rdma(send_r, recv_r, ssem_r, rsem_r, 2, k, right)
            d.start()
            descs_r[2, k] = d
            descs_l[1, k].wait_recv()
            send_l[2, k, :, :] = p_ref[rows(my - 1), colB(k)] + recv_l[1, k]
            d = rdma(send_l, recv_l, ssem_l, rsem_l, 2, k, left)
            d.start()
            descs_l[2, k] = d

        for k in range(SEG):
            descs_r[2, k].wait_recv()
            out_ref[:, colA(k)] = p_ref[rows(my), colA(k)] + recv_r[2, k]
            descs_l[2, k].wait_recv()
            out_ref[:, colB(k)] = p_ref[rows(my), colB(k)] + recv_l[2, k]

        for d in list(descs_r.values()) + list(descs_l.values()):
            d.wait_send()

    comm_shape = (N_DEV - 1, SEG, m_chunk, segw)
    sem_shape = (N_DEV - 1, SEG)
    return pl.pallas_call(
        body,
        out_shape=jax.ShapeDtypeStruct((m_chunk, n), jnp.float32),
        in_specs=[
            pl.BlockSpec(memory_space=pltpu.VMEM),
            pl.BlockSpec(memory_space=pltpu.VMEM),
        ],
        out_specs=pl.BlockSpec(memory_space=pltpu.VMEM),
        scratch_shapes=[
            pltpu.VMEM((m_full, n), jnp.float32),
            pltpu.VMEM(comm_shape, jnp.float32),
            pltpu.VMEM(comm_shape, jnp.float32),
            pltpu.VMEM(comm_shape, jnp.float32),
            pltpu.VMEM(comm_shape, jnp.float32),
            pltpu.SemaphoreType.DMA(sem_shape),
            pltpu.SemaphoreType.DMA(sem_shape),
            pltpu.SemaphoreType.DMA(sem_shape),
            pltpu.SemaphoreType.DMA(sem_shape),
        ],
        compiler_params=pltpu.CompilerParams(collective_id=0),
    )(x, w_mat)
